# Initial kernel scaffold; baseline (speedup 1.0000x reference)
import jax
import jax.numpy as jnp
from jax.experimental import pallas as pl


def kernel(x, edge_index, W1, att_src1, att_dst1, b1, W2, att_src2, att_dst2, b2):
    raise NotImplementedError("write your pallas kernel here")


# speedup vs baseline: 1.8542x; 1.8542x over previous
import functools

import jax
import jax.numpy as jnp
from jax import lax
from jax.experimental import pallas as pl

N_NODES = 10000
D = 128
BM = 1000


def _mm_body(x_ref, w_ref, o_ref):
    o_ref[...] = jnp.dot(x_ref[...], w_ref[...], preferred_element_type=jnp.float32)


def _mm(x, wc):
    m, k = x.shape
    n = wc.shape[1]
    return pl.pallas_call(
        _mm_body,
        grid=(m // BM,),
        in_specs=[
            pl.BlockSpec((BM, k), lambda i: (i, 0)),
            pl.BlockSpec((k, n), lambda i: (0, 0)),
        ],
        out_specs=pl.BlockSpec((BM, n), lambda i: (i, 0)),
        out_shape=jax.ShapeDtypeStruct((m, n), jnp.float32),
    )(x, wc)


def _norm_mm_body(a_ref, d_ref, b_ref, w_ref, o_ref):
    g = a_ref[...] / (d_ref[...] + 1e-16) + b_ref[...]
    g = jnp.maximum(g, 0.0)
    o_ref[...] = jnp.dot(g, w_ref[...], preferred_element_type=jnp.float32)


def _norm_mm(acc, denom, b, wc):
    m = acc.shape[0]
    n = wc.shape[1]
    return pl.pallas_call(
        _norm_mm_body,
        grid=(m // BM,),
        in_specs=[
            pl.BlockSpec((BM, D), lambda i: (i, 0)),
            pl.BlockSpec((BM, 1), lambda i: (i, 0)),
            pl.BlockSpec((1, D), lambda i: (0, 0)),
            pl.BlockSpec((D, n), lambda i: (0, 0)),
        ],
        out_specs=pl.BlockSpec((BM, n), lambda i: (i, 0)),
        out_shape=jax.ShapeDtypeStruct((m, n), jnp.float32),
    )(acc, denom.reshape(m, 1), b.reshape(1, D), wc)


def _norm_body(a_ref, d_ref, b_ref, o_ref):
    o_ref[...] = a_ref[...] / (d_ref[...] + 1e-16) + b_ref[...]


def _norm(acc, denom, b):
    m = acc.shape[0]
    return pl.pallas_call(
        _norm_body,
        grid=(m // BM,),
        in_specs=[
            pl.BlockSpec((BM, D), lambda i: (i, 0)),
            pl.BlockSpec((BM, 1), lambda i: (i, 0)),
            pl.BlockSpec((1, D), lambda i: (0, 0)),
        ],
        out_specs=pl.BlockSpec((BM, D), lambda i: (i, 0)),
        out_shape=jax.ShapeDtypeStruct((m, D), jnp.float32),
    )(acc, denom.reshape(m, 1), b.reshape(1, D))


def _edge_phase(h, a_src, a_dst, src, dst):
    alpha = a_src[src] + a_dst[dst]
    alpha = jnp.where(alpha > 0, alpha, 0.2 * alpha)
    w = jnp.exp(alpha)
    denom = jax.ops.segment_sum(w, dst, num_segments=N_NODES)
    acc = jax.ops.segment_sum(h[src] * w[:, None], dst, num_segments=N_NODES)
    return acc, denom


def _augment(W, att_src, att_dst):
    A = jnp.zeros((D, D), jnp.float32)
    A = A.at[:, 0].set(att_src).at[:, 1].set(att_dst)
    return jnp.concatenate([W, W @ A], axis=1)


def kernel(x, edge_index, W1, att_src1, att_dst1, b1, W2, att_src2, att_dst2, b2):
    src = edge_index[0]
    dst = edge_index[1]

    wc1 = _augment(W1, att_src1, att_dst1)
    out1 = _mm(x, wc1)
    h1, as1, ad1 = out1[:, :D], out1[:, D], out1[:, D + 1]
    acc1, den1 = _edge_phase(h1, as1, ad1, src, dst)

    wc2 = _augment(W2, att_src2, att_dst2)
    out2 = _norm_mm(acc1, den1, b1, wc2)
    h2, as2, ad2 = out2[:, :D], out2[:, D], out2[:, D + 1]
    acc2, den2 = _edge_phase(h2, as2, ad2, src, dst)

    return _norm(acc2, den2, b2)


# speedup vs baseline: 21.1183x; 11.3893x over previous
import functools

import jax
import jax.numpy as jnp
from jax import lax
from jax.experimental import pallas as pl
from jax.experimental.pallas import tpu as pltpu
from jax.experimental.pallas import tpu_sc as plsc

N_NODES = 10000
D = 128
BM = 1000

NC = 2
NS = 16
NW = NC * NS
E = 320000
EPW = E // NW
CB = 128
EPWP = 10112
NCHB = EPWP // CB
RPS = 624
TAIL = N_NODES - NS * RPS


def _mm_body(x_ref, w_ref, o_ref):
    o_ref[...] = jnp.dot(x_ref[...], w_ref[...], preferred_element_type=jnp.float32)


def _mm(x, wc):
    m, k = x.shape
    n = wc.shape[1]
    return pl.pallas_call(
        _mm_body,
        grid=(m // BM,),
        in_specs=[
            pl.BlockSpec((BM, k), lambda i: (i, 0)),
            pl.BlockSpec((k, n), lambda i: (0, 0)),
        ],
        out_specs=pl.BlockSpec((BM, n), lambda i: (i, 0)),
        out_shape=jax.ShapeDtypeStruct((m, n), jnp.float32),
    )(x, wc)


def _norm_mm_body(a_ref, d_ref, b_ref, w_ref, o_ref):
    den = jnp.sum(d_ref[...], axis=1) + 1e-16
    g = (a_ref[0] + a_ref[1]) / den[:, None] + b_ref[...]
    g = jnp.maximum(g, 0.0)
    o_ref[...] = jnp.dot(g, w_ref[...], preferred_element_type=jnp.float32)


def _norm_mm(acc2, den32, b, wc):
    m = acc2.shape[1]
    n = wc.shape[1]
    return pl.pallas_call(
        _norm_mm_body,
        grid=(m // BM,),
        in_specs=[
            pl.BlockSpec((2, BM, D), lambda i: (0, i, 0)),
            pl.BlockSpec((BM, NW), lambda i: (i, 0)),
            pl.BlockSpec((1, D), lambda i: (0, 0)),
            pl.BlockSpec((D, n), lambda i: (0, 0)),
        ],
        out_specs=pl.BlockSpec((BM, n), lambda i: (i, 0)),
        out_shape=jax.ShapeDtypeStruct((m, n), jnp.float32),
    )(acc2, den32.T, b.reshape(1, D), wc)


def _norm_body(a_ref, d_ref, b_ref, o_ref):
    den = jnp.sum(d_ref[...], axis=1) + 1e-16
    o_ref[...] = (a_ref[0] + a_ref[1]) / den[:, None] + b_ref[...]


def _norm(acc2, den32, b):
    m = acc2.shape[1]
    return pl.pallas_call(
        _norm_body,
        grid=(m // BM,),
        in_specs=[
            pl.BlockSpec((2, BM, D), lambda i: (0, i, 0)),
            pl.BlockSpec((BM, NW), lambda i: (i, 0)),
            pl.BlockSpec((1, D), lambda i: (0, 0)),
        ],
        out_specs=pl.BlockSpec((BM, D), lambda i: (i, 0)),
        out_shape=jax.ShapeDtypeStruct((m, D), jnp.float32),
    )(acc2, den32.T, b.reshape(1, D))


_SC_MESH = dict(core_axis_name="c", subcore_axis_name="s", num_cores=NC,
                num_subcores=NS)


def _attn_sc(a_src, a_dst, srcf, dstf):
    @functools.partial(
        pl.kernel,
        out_type=(
            jax.ShapeDtypeStruct((NW, 1, EPWP), jnp.float32),
            jax.ShapeDtypeStruct((NW, 1, N_NODES), jnp.float32),
        ),
        mesh=plsc.VectorSubcoreMesh(**_SC_MESH),
        scratch_types=dict(
            src1=pltpu.VMEM((EPWP,), jnp.int32),
            dst1=pltpu.VMEM((EPWP,), jnp.int32),
            asv=pltpu.VMEM((N_NODES,), jnp.float32),
            adv=pltpu.VMEM((N_NODES,), jnp.float32),
            wv=pltpu.VMEM((EPWP,), jnp.float32),
            denv=pltpu.VMEM((N_NODES,), jnp.float32),
        ),
        compiler_params=pltpu.CompilerParams(needs_layout_passes=False),
    )
    def k(asrc_hbm, adst_hbm, src_hbm, dst_hbm, w_out, den_out,
          src1, dst1, asv, adv, wv, denv):
        cid = lax.axis_index("c")
        sid = lax.axis_index("s")
        wid = sid * NC + cid

        pltpu.sync_copy(src_hbm.at[wid, 0], src1)
        pltpu.sync_copy(dst_hbm.at[wid, 0], dst1)
        pltpu.sync_copy(asrc_hbm, asv)
        pltpu.sync_copy(adst_hbm, adv)

        zeros16 = jnp.zeros((16,), jnp.float32)

        def zden(i, _):
            denv[pl.ds(i * 16, 16)] = zeros16
            return 0
        lax.fori_loop(0, N_NODES // 16, zden, 0)

        def edge16(j, _):
            sv = src1[pl.ds(j * 16, 16)]
            dv = dst1[pl.ds(j * 16, 16)]
            a = plsc.load_gather(asv, [sv]) + plsc.load_gather(adv, [dv])
            a = jnp.where(a > 0, a, a * 0.2)
            w = jnp.exp(a)
            wv[pl.ds(j * 16, 16)] = w
            plsc.addupdate_scatter(denv, [dv], w)
            return 0
        lax.fori_loop(0, EPW // 16, edge16, 0)

        for t in range((EPWP - EPW) // 16):
            wv[pl.ds(EPW + t * 16, 16)] = zeros16

        pltpu.sync_copy(wv, w_out.at[wid, 0])
        pltpu.sync_copy(denv, den_out.at[wid, 0])

    w3, den = k(a_src, a_dst, srcf, dstf)
    return w3, den.reshape(NW, N_NODES)


def _agg_sc(h, w3, srcf, dstf):
    @functools.partial(
        pl.kernel,
        out_type=jax.ShapeDtypeStruct((NC, N_NODES, D), jnp.float32),
        mesh=plsc.VectorSubcoreMesh(**_SC_MESH),
        scratch_types=dict(
            src1=pltpu.VMEM((EPWP,), jnp.int32),
            wb=pltpu.VMEM((CB,), jnp.float32),
            db=pltpu.VMEM((CB,), jnp.int32),
            rows=pltpu.VMEM((CB, D), jnp.float32),
            acc_s=pltpu.VMEM_SHARED((N_NODES, D), jnp.float32),
            sem=pltpu.SemaphoreType.DMA,
            semd=pltpu.SemaphoreType.DMA,
        ),
        compiler_params=pltpu.CompilerParams(needs_layout_passes=False),
    )
    def k(h_hbm, w_hbm, src_hbm, dst_hbm, acc_out,
          src1, wb, db, rows, acc_s, sem, semd):
        cid = lax.axis_index("c")
        sid = lax.axis_index("s")
        wid = sid * NC + cid

        pltpu.sync_copy(src_hbm.at[wid, 0], src1)

        zeros16 = jnp.zeros((16,), jnp.float32)

        for i in range(16):
            for t in range(D // 16):
                rows[i, pl.ds(t * 16, 16)] = zeros16

        def zacc(i, _):
            pltpu.sync_copy(rows.at[pl.ds(0, 16)],
                            acc_s.at[pl.ds(sid * RPS + i * 16, 16)])
            return 0
        lax.fori_loop(0, RPS // 16, zacc, 0)

        @pl.when(sid == NS - 1)
        def _():
            pltpu.sync_copy(rows.at[pl.ds(0, 16)], acc_s.at[pl.ds(NS * RPS, TAIL)])

        plsc.subcore_barrier()

        def chunk_b(j, _):
            cdb = pltpu.async_copy(dst_hbm.at[wid, 0, pl.ds(j * CB, CB)], db, semd)
            cwb = pltpu.async_copy(w_hbm.at[wid, 0, pl.ds(j * CB, CB)], wb, semd)
            pltpu.async_copy(h_hbm.at[src1.at[pl.ds(j * CB, CB)]], rows, sem).wait()
            cdb.wait()
            cwb.wait()

            def row_body(i, _):
                ws = plsc.load_gather(wb, [jnp.full((16,), 0, jnp.int32) + i])
                for t in range(D // 16):
                    rows[i, pl.ds(t * 16, 16)] = rows[i, pl.ds(t * 16, 16)] * ws
                return 0
            lax.fori_loop(0, CB, row_body, 0)

            pltpu.sync_copy(rows, acc_s.at[db], add=True)
            return 0
        lax.fori_loop(0, NCHB, chunk_b, 0)

        plsc.subcore_barrier()
        pltpu.sync_copy(acc_s.at[pl.ds(sid * RPS, RPS)],
                        acc_out.at[cid].at[pl.ds(sid * RPS, RPS)])

        @pl.when(sid == NS - 1)
        def _():
            pltpu.sync_copy(acc_s.at[pl.ds(NS * RPS, TAIL)],
                            acc_out.at[cid].at[pl.ds(NS * RPS, TAIL)])

    return k(h, w3, srcf, dstf)


def _edge_sc(h, a_src, a_dst, srcf, dstf):
    w3, den32 = _attn_sc(a_src, a_dst, srcf, dstf)
    acc2 = _agg_sc(h, w3, srcf, dstf)
    return acc2, den32


def _augment(W, att_src, att_dst):
    A = jnp.zeros((D, D), jnp.float32)
    A = A.at[:, 0].set(att_src).at[:, 1].set(att_dst)
    return jnp.concatenate([W, W @ A], axis=1)


def kernel(x, edge_index, W1, att_src1, att_dst1, b1, W2, att_src2, att_dst2, b2):
    pad = ((0, 0), (0, EPWP - EPW))
    srcf = jnp.pad(edge_index[0].reshape(NW, EPW), pad).reshape(NW, 1, EPWP)
    dstf = jnp.pad(edge_index[1].reshape(NW, EPW), pad).reshape(NW, 1, EPWP)

    wc1 = _augment(W1, att_src1, att_dst1)
    out1 = _mm(x, wc1)
    h1 = out1[:, :D]
    as1 = out1[:, D]
    ad1 = out1[:, D + 1]
    acc1, den1 = _edge_sc(h1, as1, ad1, srcf, dstf)

    wc2 = _augment(W2, att_src2, att_dst2)
    out2 = _norm_mm(acc1, den1, b1, wc2)
    h2 = out2[:, :D]
    as2 = out2[:, D]
    ad2 = out2[:, D + 1]
    acc2, den2 = _edge_sc(h2, as2, ad2, srcf, dstf)

    return _norm(acc2, den2, b2)


# speedup vs baseline: 22.2368x; 1.0530x over previous
import functools

import jax
import jax.numpy as jnp
from jax import lax
from jax.experimental import pallas as pl
from jax.experimental.pallas import tpu as pltpu
from jax.experimental.pallas import tpu_sc as plsc

N_NODES = 10000
D = 128
BM = 1000

NC = 2
NS = 16
NW = NC * NS
E = 320000
EPW = E // NW
CBH = 64
EPWP = 10112
PAIRS = EPWP // (2 * CBH)
RPS = 624
TAIL = N_NODES - NS * RPS


def _mm_body(x_ref, w_ref, o_ref):
    o_ref[...] = jnp.dot(x_ref[...], w_ref[...], preferred_element_type=jnp.float32)


def _mm(x, wc):
    m, k = x.shape
    n = wc.shape[1]
    return pl.pallas_call(
        _mm_body,
        grid=(m // BM,),
        in_specs=[
            pl.BlockSpec((BM, k), lambda i: (i, 0)),
            pl.BlockSpec((k, n), lambda i: (0, 0)),
        ],
        out_specs=pl.BlockSpec((BM, n), lambda i: (i, 0)),
        out_shape=jax.ShapeDtypeStruct((m, n), jnp.float32),
    )(x, wc)


def _norm_mm_body(a_ref, d_ref, b_ref, w_ref, o_ref):
    den = jnp.sum(d_ref[...], axis=1) + 1e-16
    g = (a_ref[0] + a_ref[1]) / den[:, None] + b_ref[...]
    g = jnp.maximum(g, 0.0)
    o_ref[...] = jnp.dot(g, w_ref[...], preferred_element_type=jnp.float32)


def _norm_mm(acc2, den32, b, wc):
    m = acc2.shape[1]
    n = wc.shape[1]
    return pl.pallas_call(
        _norm_mm_body,
        grid=(m // BM,),
        in_specs=[
            pl.BlockSpec((2, BM, D), lambda i: (0, i, 0)),
            pl.BlockSpec((BM, NW), lambda i: (i, 0)),
            pl.BlockSpec((1, D), lambda i: (0, 0)),
            pl.BlockSpec((D, n), lambda i: (0, 0)),
        ],
        out_specs=pl.BlockSpec((BM, n), lambda i: (i, 0)),
        out_shape=jax.ShapeDtypeStruct((m, n), jnp.float32),
    )(acc2, den32.T, b.reshape(1, D), wc)


def _norm_body(a_ref, d_ref, b_ref, o_ref):
    den = jnp.sum(d_ref[...], axis=1) + 1e-16
    o_ref[...] = (a_ref[0] + a_ref[1]) / den[:, None] + b_ref[...]


def _norm(acc2, den32, b):
    m = acc2.shape[1]
    return pl.pallas_call(
        _norm_body,
        grid=(m // BM,),
        in_specs=[
            pl.BlockSpec((2, BM, D), lambda i: (0, i, 0)),
            pl.BlockSpec((BM, NW), lambda i: (i, 0)),
            pl.BlockSpec((1, D), lambda i: (0, 0)),
        ],
        out_specs=pl.BlockSpec((BM, D), lambda i: (i, 0)),
        out_shape=jax.ShapeDtypeStruct((m, D), jnp.float32),
    )(acc2, den32.T, b.reshape(1, D))


_SC_MESH = dict(core_axis_name="c", subcore_axis_name="s", num_cores=NC,
                num_subcores=NS)


def _attn_sc(a_src, a_dst, srcf, dstf):
    @functools.partial(
        pl.kernel,
        out_type=(
            jax.ShapeDtypeStruct((NW, 1, EPWP), jnp.float32),
            jax.ShapeDtypeStruct((NW, 1, N_NODES), jnp.float32),
        ),
        mesh=plsc.VectorSubcoreMesh(**_SC_MESH),
        scratch_types=dict(
            src1=pltpu.VMEM((EPWP,), jnp.int32),
            dst1=pltpu.VMEM((EPWP,), jnp.int32),
            asv=pltpu.VMEM((N_NODES,), jnp.float32),
            adv=pltpu.VMEM((N_NODES,), jnp.float32),
            wv=pltpu.VMEM((EPWP,), jnp.float32),
            denv=pltpu.VMEM((N_NODES,), jnp.float32),
        ),
        compiler_params=pltpu.CompilerParams(needs_layout_passes=False),
    )
    def k(asrc_hbm, adst_hbm, src_hbm, dst_hbm, w_out, den_out,
          src1, dst1, asv, adv, wv, denv):
        cid = lax.axis_index("c")
        sid = lax.axis_index("s")
        wid = sid * NC + cid

        pltpu.sync_copy(src_hbm.at[wid, 0], src1)
        pltpu.sync_copy(dst_hbm.at[wid, 0], dst1)
        pltpu.sync_copy(asrc_hbm, asv)
        pltpu.sync_copy(adst_hbm, adv)

        zeros16 = jnp.zeros((16,), jnp.float32)

        def zden(i, _):
            denv[pl.ds(i * 16, 16)] = zeros16
            return 0
        lax.fori_loop(0, N_NODES // 16, zden, 0)

        def edge16(j, _):
            sv = src1[pl.ds(j * 16, 16)]
            dv = dst1[pl.ds(j * 16, 16)]
            a = plsc.load_gather(asv, [sv]) + plsc.load_gather(adv, [dv])
            a = jnp.where(a > 0, a, a * 0.2)
            w = jnp.exp(a)
            wv[pl.ds(j * 16, 16)] = w
            plsc.addupdate_scatter(denv, [dv], w)
            return 0
        lax.fori_loop(0, EPW // 16, edge16, 0)

        for t in range((EPWP - EPW) // 16):
            wv[pl.ds(EPW + t * 16, 16)] = zeros16

        pltpu.sync_copy(wv, w_out.at[wid, 0])
        pltpu.sync_copy(denv, den_out.at[wid, 0])

    w3, den = k(a_src, a_dst, srcf, dstf)
    return w3, den.reshape(NW, N_NODES)


def _agg_sc(h, w4, srcf, dst4):
    @functools.partial(
        pl.kernel,
        out_type=jax.ShapeDtypeStruct((NC, N_NODES, D), jnp.float32),
        mesh=plsc.VectorSubcoreMesh(**_SC_MESH),
        scratch_types=dict(
            src1=pltpu.VMEM((EPWP,), jnp.int32),
            db2=pltpu.VMEM((2, CBH), jnp.int32),
            wb2=pltpu.VMEM((2, CBH), jnp.float32),
            r0=pltpu.VMEM((CBH, D), jnp.float32),
            r1=pltpu.VMEM((CBH, D), jnp.float32),
            acc_s=pltpu.VMEM_SHARED((N_NODES, D), jnp.float32),
            sdw=pltpu.SemaphoreType.DMA,
            sg0=pltpu.SemaphoreType.DMA,
            sg1=pltpu.SemaphoreType.DMA,
            ss0=pltpu.SemaphoreType.DMA,
            ss1=pltpu.SemaphoreType.DMA,
        ),
        compiler_params=pltpu.CompilerParams(needs_layout_passes=False),
    )
    def k(h_hbm, w_hbm, src_hbm, dst_hbm, acc_out,
          src1, db2, wb2, r0, r1, acc_s, sdw, sg0, sg1, ss0, ss1):
        cid = lax.axis_index("c")
        sid = lax.axis_index("s")
        wid = sid * NC + cid

        pltpu.sync_copy(src_hbm.at[wid, 0], src1)

        zeros16 = jnp.zeros((16,), jnp.float32)

        for i in range(16):
            for t in range(D // 16):
                r0[i, pl.ds(t * 16, 16)] = zeros16

        def zacc(i, _):
            pltpu.sync_copy(r0.at[pl.ds(0, 16)],
                            acc_s.at[pl.ds(sid * RPS + i * 16, 16)])
            return 0
        lax.fori_loop(0, RPS // 16, zacc, 0)

        @pl.when(sid == NS - 1)
        def _():
            pltpu.sync_copy(r0.at[pl.ds(0, 16)], acc_s.at[pl.ds(NS * RPS, TAIL)])

        plsc.subcore_barrier()

        def scale(rbuf, half):
            def row_body(i, _):
                ws = plsc.load_gather(
                    wb2, [jnp.full((16,), half, jnp.int32),
                          jnp.full((16,), 0, jnp.int32) + i])
                for t in range(D // 16):
                    rbuf[i, pl.ds(t * 16, 16)] = rbuf[i, pl.ds(t * 16, 16)] * ws
                return 0
            lax.fori_loop(0, CBH, row_body, 0)

        def pair_body(t, _):
            base = t * 2 * CBH
            cd = pltpu.async_copy(dst_hbm.at[wid, t], db2, sdw)
            cw = pltpu.async_copy(w_hbm.at[wid, t], wb2, sdw)
            g0 = pltpu.async_copy(h_hbm.at[src1.at[pl.ds(base, CBH)]], r0, sg0)
            g1 = pltpu.async_copy(h_hbm.at[src1.at[pl.ds(base + CBH, CBH)]], r1, sg1)
            g0.wait()
            cd.wait()
            cw.wait()
            scale(r0, 0)
            s0 = pltpu.async_copy(r0, acc_s.at[db2.at[0]], ss0, add=True)
            g1.wait()
            scale(r1, 1)
            s1 = pltpu.async_copy(r1, acc_s.at[db2.at[1]], ss1, add=True)
            s0.wait()
            s1.wait()
            return 0
        lax.fori_loop(0, PAIRS, pair_body, 0)

        plsc.subcore_barrier()
        pltpu.sync_copy(acc_s.at[pl.ds(sid * RPS, RPS)],
                        acc_out.at[cid].at[pl.ds(sid * RPS, RPS)])

        @pl.when(sid == NS - 1)
        def _():
            pltpu.sync_copy(acc_s.at[pl.ds(NS * RPS, TAIL)],
                            acc_out.at[cid].at[pl.ds(NS * RPS, TAIL)])

    return k(h, w4, srcf, dst4)


def _edge_sc(h, a_src, a_dst, srcf, dstf, dst4):
    w3, den32 = _attn_sc(a_src, a_dst, srcf, dstf)
    w4 = w3.reshape(NW, PAIRS, 2, CBH)
    acc2 = _agg_sc(h, w4, srcf, dst4)
    return acc2, den32


def _augment(W, att_src, att_dst):
    A = jnp.zeros((D, D), jnp.float32)
    A = A.at[:, 0].set(att_src).at[:, 1].set(att_dst)
    return jnp.concatenate([W, W @ A], axis=1)


def kernel(x, edge_index, W1, att_src1, att_dst1, b1, W2, att_src2, att_dst2, b2):
    pad = ((0, 0), (0, EPWP - EPW))
    srcf = jnp.pad(edge_index[0].reshape(NW, EPW), pad).reshape(NW, 1, EPWP)
    dstf = jnp.pad(edge_index[1].reshape(NW, EPW), pad).reshape(NW, 1, EPWP)
    dst4 = dstf.reshape(NW, PAIRS, 2, CBH)

    wc1 = _augment(W1, att_src1, att_dst1)
    out1 = _mm(x, wc1)
    h1 = out1[:, :D]
    as1 = out1[:, D]
    ad1 = out1[:, D + 1]
    acc1, den1 = _edge_sc(h1, as1, ad1, srcf, dstf, dst4)

    wc2 = _augment(W2, att_src2, att_dst2)
    out2 = _norm_mm(acc1, den1, b1, wc2)
    h2 = out2[:, :D]
    as2 = out2[:, D]
    ad2 = out2[:, D + 1]
    acc2, den2 = _edge_sc(h2, as2, ad2, srcf, dstf, dst4)

    return _norm(acc2, den2, b2)


# speedup vs baseline: 24.7555x; 1.1133x over previous
import functools

import jax
import jax.numpy as jnp
from jax import lax
from jax.experimental import pallas as pl
from jax.experimental.pallas import tpu as pltpu
from jax.experimental.pallas import tpu_sc as plsc

N_NODES = 10000
D = 128
BM = 1000

NC = 2
NS = 16
NW = NC * NS
E = 320000
EPW = E // NW
CBH = 64
EPWP = 10112
PAIRS = EPWP // (2 * CBH)
RPS = 624
TAIL = N_NODES - NS * RPS


def _mm_body(x_ref, w_ref, o_ref):
    o_ref[...] = jnp.dot(x_ref[...], w_ref[...], preferred_element_type=jnp.float32)


def _mm(x, wc):
    m, k = x.shape
    n = wc.shape[1]
    return pl.pallas_call(
        _mm_body,
        grid=(m // BM,),
        in_specs=[
            pl.BlockSpec((BM, k), lambda i: (i, 0)),
            pl.BlockSpec((k, n), lambda i: (0, 0)),
        ],
        out_specs=pl.BlockSpec((BM, n), lambda i: (i, 0)),
        out_shape=jax.ShapeDtypeStruct((m, n), jnp.float32),
    )(x, wc)


def _norm_mm_body(a_ref, d_ref, b_ref, w_ref, o_ref):
    den = jnp.sum(d_ref[...], axis=1) + 1e-16
    g = (a_ref[0] + a_ref[1]) / den[:, None] + b_ref[...]
    g = jnp.maximum(g, 0.0)
    o_ref[...] = jnp.dot(g, w_ref[...], preferred_element_type=jnp.float32)


def _norm_mm(acc2, den32, b, wc):
    m = acc2.shape[1]
    n = wc.shape[1]
    return pl.pallas_call(
        _norm_mm_body,
        grid=(m // BM,),
        in_specs=[
            pl.BlockSpec((2, BM, D), lambda i: (0, i, 0)),
            pl.BlockSpec((BM, NW), lambda i: (i, 0)),
            pl.BlockSpec((1, D), lambda i: (0, 0)),
            pl.BlockSpec((D, n), lambda i: (0, 0)),
        ],
        out_specs=pl.BlockSpec((BM, n), lambda i: (i, 0)),
        out_shape=jax.ShapeDtypeStruct((m, n), jnp.float32),
    )(acc2, den32.T, b.reshape(1, D), wc)


def _norm_body(a_ref, d_ref, b_ref, o_ref):
    den = jnp.sum(d_ref[...], axis=1) + 1e-16
    o_ref[...] = (a_ref[0] + a_ref[1]) / den[:, None] + b_ref[...]


def _norm(acc2, den32, b):
    m = acc2.shape[1]
    return pl.pallas_call(
        _norm_body,
        grid=(m // BM,),
        in_specs=[
            pl.BlockSpec((2, BM, D), lambda i: (0, i, 0)),
            pl.BlockSpec((BM, NW), lambda i: (i, 0)),
            pl.BlockSpec((1, D), lambda i: (0, 0)),
        ],
        out_specs=pl.BlockSpec((BM, D), lambda i: (i, 0)),
        out_shape=jax.ShapeDtypeStruct((m, D), jnp.float32),
    )(acc2, den32.T, b.reshape(1, D))


_SC_MESH = dict(core_axis_name="c", subcore_axis_name="s", num_cores=NC,
                num_subcores=NS)


def _attn_sc(a_src, a_dst, srcf, dstf):
    @functools.partial(
        pl.kernel,
        out_type=(
            jax.ShapeDtypeStruct((NW, 1, EPWP), jnp.float32),
            jax.ShapeDtypeStruct((NW, 1, N_NODES), jnp.float32),
        ),
        mesh=plsc.VectorSubcoreMesh(**_SC_MESH),
        scratch_types=dict(
            src1=pltpu.VMEM((EPWP,), jnp.int32),
            dst1=pltpu.VMEM((EPWP,), jnp.int32),
            asv=pltpu.VMEM((N_NODES,), jnp.float32),
            adv=pltpu.VMEM((N_NODES,), jnp.float32),
            wv=pltpu.VMEM((EPWP,), jnp.float32),
            denv=pltpu.VMEM((N_NODES,), jnp.float32),
        ),
        compiler_params=pltpu.CompilerParams(needs_layout_passes=False),
    )
    def k(asrc_hbm, adst_hbm, src_hbm, dst_hbm, w_out, den_out,
          src1, dst1, asv, adv, wv, denv):
        cid = lax.axis_index("c")
        sid = lax.axis_index("s")
        wid = sid * NC + cid

        pltpu.sync_copy(src_hbm.at[wid, 0], src1)
        pltpu.sync_copy(dst_hbm.at[wid, 0], dst1)
        pltpu.sync_copy(asrc_hbm, asv)
        pltpu.sync_copy(adst_hbm, adv)

        zeros16 = jnp.zeros((16,), jnp.float32)

        def zden(i, _):
            denv[pl.ds(i * 16, 16)] = zeros16
            return 0
        lax.fori_loop(0, N_NODES // 16, zden, 0)

        def edge16(j, _):
            sv = src1[pl.ds(j * 16, 16)]
            dv = dst1[pl.ds(j * 16, 16)]
            a = plsc.load_gather(asv, [sv]) + plsc.load_gather(adv, [dv])
            a = jnp.where(a > 0, a, a * 0.2)
            w = jnp.exp(a)
            wv[pl.ds(j * 16, 16)] = w
            plsc.addupdate_scatter(denv, [dv], w)
            return 0
        lax.fori_loop(0, EPW // 16, edge16, 0)

        for t in range((EPWP - EPW) // 16):
            wv[pl.ds(EPW + t * 16, 16)] = zeros16

        pltpu.sync_copy(wv, w_out.at[wid, 0])
        pltpu.sync_copy(denv, den_out.at[wid, 0])

    w3, den = k(a_src, a_dst, srcf, dstf)
    return w3, den.reshape(NW, N_NODES)


def _agg_sc(h, w4, srcf, dst4):
    @functools.partial(
        pl.kernel,
        out_type=jax.ShapeDtypeStruct((NC, N_NODES, D), jnp.float32),
        mesh=plsc.VectorSubcoreMesh(**_SC_MESH),
        scratch_types=dict(
            src1=pltpu.VMEM((EPWP,), jnp.int32),
            db2=pltpu.VMEM((2, CBH), jnp.int32),
            wb2=pltpu.VMEM((2, CBH), jnp.float32),
            r0=pltpu.VMEM((CBH, D), jnp.float32),
            r1=pltpu.VMEM((CBH, D), jnp.float32),
            acc_s=pltpu.VMEM_SHARED((N_NODES, D), jnp.float32),
            sdw=pltpu.SemaphoreType.DMA,
            sg0=pltpu.SemaphoreType.DMA,
            sg1=pltpu.SemaphoreType.DMA,
            ss0=pltpu.SemaphoreType.DMA,
            ss1=pltpu.SemaphoreType.DMA,
        ),
        compiler_params=pltpu.CompilerParams(needs_layout_passes=False),
    )
    def k(h_hbm, w_hbm, src_hbm, dst_hbm, acc_out,
          src1, db2, wb2, r0, r1, acc_s, sdw, sg0, sg1, ss0, ss1):
        cid = lax.axis_index("c")
        sid = lax.axis_index("s")
        wid = sid * NC + cid

        pltpu.sync_copy(src_hbm.at[wid, 0], src1)

        zeros16 = jnp.zeros((16,), jnp.float32)

        for i in range(16):
            for t in range(D // 16):
                r0[i, pl.ds(t * 16, 16)] = zeros16

        def zacc(i, _):
            pltpu.sync_copy(r0.at[pl.ds(0, 16)],
                            acc_s.at[pl.ds(sid * RPS + i * 16, 16)])
            return 0
        lax.fori_loop(0, RPS // 16, zacc, 0)

        @pl.when(sid == NS - 1)
        def _():
            pltpu.sync_copy(r0.at[pl.ds(0, 16)], acc_s.at[pl.ds(NS * RPS, TAIL)])

        plsc.subcore_barrier()

        def scale(rbuf, half):
            for g in range(CBH // 16):
                wv16 = wb2[half, pl.ds(g * 16, 16)]
                for r in range(16):
                    ws = jnp.full((16,), wv16[r], jnp.float32)
                    row = g * 16 + r
                    for t in range(D // 16):
                        rbuf[row, pl.ds(t * 16, 16)] = (
                            rbuf[row, pl.ds(t * 16, 16)] * ws)

        def pair_body(t, _):
            base = t * 2 * CBH
            cd = pltpu.async_copy(dst_hbm.at[wid, t], db2, sdw)
            cw = pltpu.async_copy(w_hbm.at[wid, t], wb2, sdw)
            g0 = pltpu.async_copy(h_hbm.at[src1.at[pl.ds(base, CBH)]], r0, sg0)
            g1 = pltpu.async_copy(h_hbm.at[src1.at[pl.ds(base + CBH, CBH)]], r1, sg1)
            g0.wait()
            cd.wait()
            cw.wait()
            scale(r0, 0)
            s0 = pltpu.async_copy(r0, acc_s.at[db2.at[0]], ss0, add=True)
            g1.wait()
            scale(r1, 1)
            s1 = pltpu.async_copy(r1, acc_s.at[db2.at[1]], ss1, add=True)
            s0.wait()
            s1.wait()
            return 0
        lax.fori_loop(0, PAIRS, pair_body, 0)

        plsc.subcore_barrier()
        pltpu.sync_copy(acc_s.at[pl.ds(sid * RPS, RPS)],
                        acc_out.at[cid].at[pl.ds(sid * RPS, RPS)])

        @pl.when(sid == NS - 1)
        def _():
            pltpu.sync_copy(acc_s.at[pl.ds(NS * RPS, TAIL)],
                            acc_out.at[cid].at[pl.ds(NS * RPS, TAIL)])

    return k(h, w4, srcf, dst4)


def _edge_sc(h, a_src, a_dst, srcf, dstf, dst4):
    w3, den32 = _attn_sc(a_src, a_dst, srcf, dstf)
    w4 = w3.reshape(NW, PAIRS, 2, CBH)
    acc2 = _agg_sc(h, w4, srcf, dst4)
    return acc2, den32


def _augment(W, att_src, att_dst):
    A = jnp.zeros((D, D), jnp.float32)
    A = A.at[:, 0].set(att_src).at[:, 1].set(att_dst)
    return jnp.concatenate([W, W @ A], axis=1)


def kernel(x, edge_index, W1, att_src1, att_dst1, b1, W2, att_src2, att_dst2, b2):
    pad = ((0, 0), (0, EPWP - EPW))
    srcf = jnp.pad(edge_index[0].reshape(NW, EPW), pad).reshape(NW, 1, EPWP)
    dstf = jnp.pad(edge_index[1].reshape(NW, EPW), pad).reshape(NW, 1, EPWP)
    dst4 = dstf.reshape(NW, PAIRS, 2, CBH)

    wc1 = _augment(W1, att_src1, att_dst1)
    out1 = _mm(x, wc1)
    h1 = out1[:, :D]
    as1 = out1[:, D]
    ad1 = out1[:, D + 1]
    acc1, den1 = _edge_sc(h1, as1, ad1, srcf, dstf, dst4)

    wc2 = _augment(W2, att_src2, att_dst2)
    out2 = _norm_mm(acc1, den1, b1, wc2)
    h2 = out2[:, :D]
    as2 = out2[:, D]
    ad2 = out2[:, D + 1]
    acc2, den2 = _edge_sc(h2, as2, ad2, srcf, dstf, dst4)

    return _norm(acc2, den2, b2)


# speedup vs baseline: 33.7102x; 1.3617x over previous
import functools

import jax
import jax.numpy as jnp
from jax import lax
from jax.experimental import pallas as pl
from jax.experimental.pallas import tpu as pltpu
from jax.experimental.pallas import tpu_sc as plsc

N_NODES = 10000
D = 128
BM = 1000

NC = 2
NS = 16
NW = NC * NS
E = 320000
EPW = E // NW
CBH = 64
EPWP = 10112
PAIRS = EPWP // (2 * CBH)
RPS = 624
TAIL = N_NODES - NS * RPS


def _mm_body(x_ref, w_ref, o_ref):
    o_ref[...] = jnp.dot(x_ref[...], w_ref[...], preferred_element_type=jnp.float32)


def _mm(x, wc):
    m, k = x.shape
    n = wc.shape[1]
    return pl.pallas_call(
        _mm_body,
        grid=(m // BM,),
        in_specs=[
            pl.BlockSpec((BM, k), lambda i: (i, 0)),
            pl.BlockSpec((k, n), lambda i: (0, 0)),
        ],
        out_specs=pl.BlockSpec((BM, n), lambda i: (i, 0)),
        out_shape=jax.ShapeDtypeStruct((m, n), jnp.float32),
    )(x, wc)


def _norm_mm_body(a_ref, d_ref, b_ref, w_ref, o_ref):
    den = jnp.sum(d_ref[...], axis=1) + 1e-16
    g = (a_ref[0] + a_ref[1]) / den[:, None] + b_ref[...]
    g = jnp.maximum(g, 0.0)
    o_ref[...] = jnp.dot(g, w_ref[...], preferred_element_type=jnp.float32)


def _norm_mm(acc2, den32, b, wc):
    m = acc2.shape[1]
    n = wc.shape[1]
    return pl.pallas_call(
        _norm_mm_body,
        grid=(m // BM,),
        in_specs=[
            pl.BlockSpec((2, BM, D), lambda i: (0, i, 0)),
            pl.BlockSpec((BM, NW), lambda i: (i, 0)),
            pl.BlockSpec((1, D), lambda i: (0, 0)),
            pl.BlockSpec((D, n), lambda i: (0, 0)),
        ],
        out_specs=pl.BlockSpec((BM, n), lambda i: (i, 0)),
        out_shape=jax.ShapeDtypeStruct((m, n), jnp.float32),
    )(acc2, den32.T, b.reshape(1, D), wc)


def _norm_body(a_ref, d_ref, b_ref, o_ref):
    den = jnp.sum(d_ref[...], axis=1) + 1e-16
    o_ref[...] = (a_ref[0] + a_ref[1]) / den[:, None] + b_ref[...]


def _norm(acc2, den32, b):
    m = acc2.shape[1]
    return pl.pallas_call(
        _norm_body,
        grid=(m // BM,),
        in_specs=[
            pl.BlockSpec((2, BM, D), lambda i: (0, i, 0)),
            pl.BlockSpec((BM, NW), lambda i: (i, 0)),
            pl.BlockSpec((1, D), lambda i: (0, 0)),
        ],
        out_specs=pl.BlockSpec((BM, D), lambda i: (i, 0)),
        out_shape=jax.ShapeDtypeStruct((m, D), jnp.float32),
    )(acc2, den32.T, b.reshape(1, D))


_SC_MESH = dict(core_axis_name="c", subcore_axis_name="s", num_cores=NC,
                num_subcores=NS)


def _attn_sc(a_src, a_dst, srcf, dstf):
    @functools.partial(
        pl.kernel,
        out_type=(
            jax.ShapeDtypeStruct((NW, 1, EPWP), jnp.float32),
            jax.ShapeDtypeStruct((NW, 1, N_NODES), jnp.float32),
        ),
        mesh=plsc.VectorSubcoreMesh(**_SC_MESH),
        scratch_types=dict(
            src1=pltpu.VMEM((EPWP,), jnp.int32),
            dst1=pltpu.VMEM((EPWP,), jnp.int32),
            asv=pltpu.VMEM((N_NODES,), jnp.float32),
            adv=pltpu.VMEM((N_NODES,), jnp.float32),
            wv=pltpu.VMEM((EPWP,), jnp.float32),
            denv=pltpu.VMEM((N_NODES,), jnp.float32),
        ),
        compiler_params=pltpu.CompilerParams(needs_layout_passes=False),
    )
    def k(asrc_hbm, adst_hbm, src_hbm, dst_hbm, w_out, den_out,
          src1, dst1, asv, adv, wv, denv):
        cid = lax.axis_index("c")
        sid = lax.axis_index("s")
        wid = sid * NC + cid

        pltpu.sync_copy(src_hbm.at[wid, 0], src1)
        pltpu.sync_copy(dst_hbm.at[wid, 0], dst1)
        pltpu.sync_copy(asrc_hbm, asv)
        pltpu.sync_copy(adst_hbm, adv)

        zeros16 = jnp.zeros((16,), jnp.float32)

        def zden(i, _):
            denv[pl.ds(i * 16, 16)] = zeros16
            return 0
        lax.fori_loop(0, N_NODES // 16, zden, 0)

        def edge16(j, _):
            sv = src1[pl.ds(j * 16, 16)]
            dv = dst1[pl.ds(j * 16, 16)]
            a = plsc.load_gather(asv, [sv]) + plsc.load_gather(adv, [dv])
            a = jnp.where(a > 0, a, a * 0.2)
            w = jnp.exp(a)
            wv[pl.ds(j * 16, 16)] = w
            plsc.addupdate_scatter(denv, [dv], w)
            return 0
        lax.fori_loop(0, EPW // 16, edge16, 0)

        for t in range((EPWP - EPW) // 16):
            wv[pl.ds(EPW + t * 16, 16)] = zeros16

        pltpu.sync_copy(wv, w_out.at[wid, 0])
        pltpu.sync_copy(denv, den_out.at[wid, 0])

    w3, den = k(a_src, a_dst, srcf, dstf)
    return w3, den.reshape(NW, N_NODES)


def _agg_sc(h, w4, srcf, dst4):
    @functools.partial(
        pl.kernel,
        out_type=jax.ShapeDtypeStruct((NC, N_NODES, D), jnp.float32),
        mesh=plsc.VectorSubcoreMesh(**_SC_MESH),
        scratch_types=dict(
            src1=pltpu.VMEM((EPWP,), jnp.int32),
            db2=pltpu.VMEM((2, CBH), jnp.int32),
            wb2=pltpu.VMEM((2, CBH), jnp.float32),
            r0=pltpu.VMEM((CBH, D // 2), jnp.float32),
            r1=pltpu.VMEM((CBH, D // 2), jnp.float32),
            ro0=pltpu.VMEM((CBH, D), jnp.float32),
            ro1=pltpu.VMEM((CBH, D), jnp.float32),
            acc_s=pltpu.VMEM_SHARED((N_NODES, D), jnp.float32),
            sdw=pltpu.SemaphoreType.DMA,
            sg0=pltpu.SemaphoreType.DMA,
            sg1=pltpu.SemaphoreType.DMA,
            ss0=pltpu.SemaphoreType.DMA,
            ss1=pltpu.SemaphoreType.DMA,
        ),
        compiler_params=pltpu.CompilerParams(needs_layout_passes=False, use_tc_tiling_on_sc=False),
    )
    def k(h_hbm, w_hbm, src_hbm, dst_hbm, acc_out,
          src1, db2, wb2, r0, r1, ro0, ro1, acc_s, sdw, sg0, sg1, ss0, ss1):
        cid = lax.axis_index("c")
        sid = lax.axis_index("s")
        wid = sid * NC + cid

        pltpu.sync_copy(src_hbm.at[wid, 0], src1)

        zeros16 = jnp.zeros((16,), jnp.float32)

        for i in range(16):
            for t in range(D // 16):
                ro0[i, pl.ds(t * 16, 16)] = zeros16

        def zacc(i, _):
            pltpu.sync_copy(ro0.at[pl.ds(0, 16)],
                            acc_s.at[pl.ds(sid * RPS + i * 16, 16)])
            return 0
        lax.fori_loop(0, RPS // 16, zacc, 0)

        @pl.when(sid == NS - 1)
        def _():
            pltpu.sync_copy(ro0.at[pl.ds(0, 16)], acc_s.at[pl.ds(NS * RPS, TAIL)])

        plsc.subcore_barrier()

        def scale(rp, ro, half):
            for g in range(CBH // 16):
                wv16 = wb2[half, pl.ds(g * 16, 16)]
                for r in range(16):
                    ws = jnp.full((16,), wv16[r], jnp.float32)
                    row = g * 16 + r
                    for t in range(D // 32):
                        v = rp[row, pl.ds(t * 16, 16)]
                        a, b = plsc.unpack(plsc.bitcast(v, jnp.bfloat16),
                                           format=plsc.PackFormat.INTERLEAVED)
                        ro[row, pl.ds(t * 32, 16)] = a * ws
                        ro[row, pl.ds(t * 32 + 16, 16)] = b * ws

        def pair_body(t, _):
            base = t * 2 * CBH
            cd = pltpu.async_copy(dst_hbm.at[wid, t], db2, sdw)
            cw = pltpu.async_copy(w_hbm.at[wid, t], wb2, sdw)
            g0 = pltpu.async_copy(h_hbm.at[src1.at[pl.ds(base, CBH)]], r0, sg0)
            g1 = pltpu.async_copy(h_hbm.at[src1.at[pl.ds(base + CBH, CBH)]], r1, sg1)
            g0.wait()
            cd.wait()
            cw.wait()
            scale(r0, ro0, 0)
            s0 = pltpu.async_copy(ro0, acc_s.at[db2.at[0]], ss0, add=True)
            g1.wait()
            scale(r1, ro1, 1)
            s1 = pltpu.async_copy(ro1, acc_s.at[db2.at[1]], ss1, add=True)
            s0.wait()
            s1.wait()
            return 0
        lax.fori_loop(0, PAIRS, pair_body, 0)

        plsc.subcore_barrier()
        pltpu.sync_copy(acc_s.at[pl.ds(sid * RPS, RPS)],
                        acc_out.at[cid].at[pl.ds(sid * RPS, RPS)])

        @pl.when(sid == NS - 1)
        def _():
            pltpu.sync_copy(acc_s.at[pl.ds(NS * RPS, TAIL)],
                            acc_out.at[cid].at[pl.ds(NS * RPS, TAIL)])

    return k(h, w4, srcf, dst4)


def _pack_h(h):
    n = h.shape[0]
    hb = h.astype(jnp.bfloat16).reshape(n, D // 32, 2, 16)
    st = jnp.stack([hb[:, :, 0, :], hb[:, :, 1, :]], axis=-1)
    return lax.bitcast_convert_type(st, jnp.float32).reshape(n, D // 2)


def _edge_sc(h, a_src, a_dst, srcf, dstf, dst4):
    w3, den32 = _attn_sc(a_src, a_dst, srcf, dstf)
    w4 = w3.reshape(NW, PAIRS, 2, CBH)
    acc2 = _agg_sc(_pack_h(h), w4, srcf, dst4)
    return acc2, den32


def _augment(W, att_src, att_dst):
    A = jnp.zeros((D, D), jnp.float32)
    A = A.at[:, 0].set(att_src).at[:, 1].set(att_dst)
    return jnp.concatenate([W, W @ A], axis=1)


def kernel(x, edge_index, W1, att_src1, att_dst1, b1, W2, att_src2, att_dst2, b2):
    pad = ((0, 0), (0, EPWP - EPW))
    srcf = jnp.pad(edge_index[0].reshape(NW, EPW), pad).reshape(NW, 1, EPWP)
    dstf = jnp.pad(edge_index[1].reshape(NW, EPW), pad).reshape(NW, 1, EPWP)
    dst4 = dstf.reshape(NW, PAIRS, 2, CBH)

    wc1 = _augment(W1, att_src1, att_dst1)
    out1 = _mm(x, wc1)
    h1 = out1[:, :D]
    as1 = out1[:, D]
    ad1 = out1[:, D + 1]
    acc1, den1 = _edge_sc(h1, as1, ad1, srcf, dstf, dst4)

    wc2 = _augment(W2, att_src2, att_dst2)
    out2 = _norm_mm(acc1, den1, b1, wc2)
    h2 = out2[:, :D]
    as2 = out2[:, D]
    ad2 = out2[:, D + 1]
    acc2, den2 = _edge_sc(h2, as2, ad2, srcf, dstf, dst4)

    return _norm(acc2, den2, b2)
